# baseline (speedup 1.0000x reference)
import functools

import jax
import jax.numpy as jnp
from jax import lax
from jax.experimental import pallas as pl
from jax.experimental.pallas import tpu as pltpu
from jax.experimental.pallas import tpu_sc as plsc

B = 16384
EMB = 128
H0, H1, H2 = 1024, 1024, 512
NC, NS = 2, 16
NW = NC * NS
CHS = (2048, 14336)
BM = 1024


def _sc_gather_chunk(user, item, user_table, item_table, off, chb):
    f32 = jnp.float32
    rpt = chb // NW
    mesh = plsc.VectorSubcoreMesh(core_axis_name="c", subcore_axis_name="s")
    out_type = (
        jax.ShapeDtypeStruct((chb, EMB), f32),
        jax.ShapeDtypeStruct((chb, EMB), f32),
    )

    @functools.partial(
        pl.kernel, mesh=mesh, out_type=out_type,
        scratch_types=[
            pltpu.VMEM((rpt,), jnp.int32),
            pltpu.VMEM((rpt,), jnp.int32),
            pltpu.VMEM((rpt, EMB), f32),
            pltpu.VMEM((rpt, EMB), f32),
            pltpu.SemaphoreType.DMA,
            pltpu.SemaphoreType.DMA,
            pltpu.SemaphoreType.DMA,
        ],
    )
    def k(u_hbm, i_hbm, ut_hbm, it_hbm, uo_hbm, io_hbm,
          uidx_v, iidx_v, urows_v, irows_v, gsem, osem0, osem1):
        wid = lax.axis_index("s") * NC + lax.axis_index("c")
        src = off + wid * rpt
        dst = wid * rpt
        pltpu.sync_copy(u_hbm.at[pl.ds(src, rpt)], uidx_v)
        pltpu.sync_copy(i_hbm.at[pl.ds(src, rpt)], iidx_v)
        pltpu.async_copy(ut_hbm.at[uidx_v], urows_v, gsem).wait()
        ou = pltpu.async_copy(urows_v, uo_hbm.at[pl.ds(dst, rpt)], osem0)
        pltpu.async_copy(it_hbm.at[iidx_v], irows_v, gsem).wait()
        oi = pltpu.async_copy(irows_v, io_hbm.at[pl.ds(dst, rpt)], osem1)
        ou.wait()
        oi.wait()

    return k(user, item, user_table, item_table)


def _sc_gather_chunk0_bias(user, item, user_table, item_table,
                           ub_flat, ib_flat, chb):
    f32 = jnp.float32
    rpt = chb // NW
    BPW = B // NW
    mesh = plsc.VectorSubcoreMesh(core_axis_name="c", subcore_axis_name="s")
    out_type = (
        jax.ShapeDtypeStruct((chb, EMB), f32),
        jax.ShapeDtypeStruct((chb, EMB), f32),
        jax.ShapeDtypeStruct((B,), f32),
        jax.ShapeDtypeStruct((B,), f32),
    )

    @functools.partial(
        pl.kernel, mesh=mesh, out_type=out_type,
        scratch_types=[
            pltpu.VMEM((rpt,), jnp.int32),
            pltpu.VMEM((rpt,), jnp.int32),
            pltpu.VMEM((rpt, EMB), f32),
            pltpu.VMEM((rpt, EMB), f32),
            pltpu.VMEM((BPW,), jnp.int32),
            pltpu.VMEM((BPW,), jnp.int32),
            pltpu.VMEM((BPW,), f32),
            pltpu.VMEM((BPW,), f32),
            pltpu.SemaphoreType.DMA,
            pltpu.SemaphoreType.DMA,
            pltpu.SemaphoreType.DMA,
            pltpu.SemaphoreType.DMA,
            pltpu.SemaphoreType.DMA,
        ],
    )
    def k(u_hbm, i_hbm, ut_hbm, it_hbm, ub_hbm, ib_hbm,
          uo_hbm, io_hbm, ubo_hbm, ibo_hbm,
          uidx_v, iidx_v, urows_v, irows_v,
          buidx_v, biidx_v, ubvals_v, ibvals_v,
          gsem, osem0, osem1, bsem0, bsem1):
        wid = lax.axis_index("s") * NC + lax.axis_index("c")
        src = wid * rpt
        base = wid * BPW
        pltpu.sync_copy(u_hbm.at[pl.ds(base, BPW)], buidx_v)
        pltpu.sync_copy(i_hbm.at[pl.ds(base, BPW)], biidx_v)
        bu = pltpu.async_copy(ub_hbm.at[buidx_v], ubvals_v, bsem0)
        bi = pltpu.async_copy(ib_hbm.at[biidx_v], ibvals_v, bsem1)
        pltpu.sync_copy(u_hbm.at[pl.ds(src, rpt)], uidx_v)
        pltpu.sync_copy(i_hbm.at[pl.ds(src, rpt)], iidx_v)
        pltpu.async_copy(ut_hbm.at[uidx_v], urows_v, gsem).wait()
        ou = pltpu.async_copy(urows_v, uo_hbm.at[pl.ds(src, rpt)], osem0)
        pltpu.async_copy(it_hbm.at[iidx_v], irows_v, gsem).wait()
        oi = pltpu.async_copy(irows_v, io_hbm.at[pl.ds(src, rpt)], osem1)
        bu.wait()
        ou2 = pltpu.async_copy(ubvals_v, ubo_hbm.at[pl.ds(base, BPW)], bsem0)
        bi.wait()
        bi2 = pltpu.async_copy(ibvals_v, ibo_hbm.at[pl.ds(base, BPW)], bsem1)
        ou.wait()
        oi.wait()
        ou2.wait()
        bi2.wait()

    return k(user, item, user_table, item_table, ub_flat, ib_flat)


def _mlp_body(u_ref, i_ref, ub_ref, ib_ref, w0_ref, b0_ref, w1_ref, b1_ref,
              w2_ref, b2_ref, o_ref):
    f32 = jnp.float32
    w0 = w0_ref[...]
    w0u, w0i = w0[:EMB], w0[EMB:]
    acc = jnp.dot(u_ref[...], w0u, preferred_element_type=f32)
    acc = acc + jnp.dot(i_ref[...], w0i, preferred_element_type=f32)
    su = jnp.sum(w0u, axis=0, keepdims=True)
    si = jnp.sum(w0i, axis=0, keepdims=True)
    ub = ub_ref[...].reshape(BM, 1)
    ib = ib_ref[...].reshape(BM, 1)
    acc = acc + b0_ref[...] + ub * su + ib * si
    h = jnp.maximum(acc, 0.0)
    h = jnp.maximum(
        jnp.dot(h, w1_ref[...], preferred_element_type=f32) + b1_ref[...],
        0.0)
    o_ref[...] = jnp.maximum(
        jnp.dot(h, w2_ref[...], preferred_element_type=f32) + b2_ref[...],
        0.0)


def _mlp_chunk(off, chb, out_buf, u_emb, i_emb, ubg, ibg,
               W0, b0, W1, b1, W2, b2):
    noff = off // BM
    in_specs = [
        pl.BlockSpec((BM, EMB), lambda i: (i, 0)),
        pl.BlockSpec((BM, EMB), lambda i: (i, 0)),
        pl.BlockSpec((BM,), lambda i: (i + noff,)),
        pl.BlockSpec((BM,), lambda i: (i + noff,)),
        pl.BlockSpec((2 * EMB, H0), lambda i: (0, 0)),
        pl.BlockSpec((1, H0), lambda i: (0, 0)),
        pl.BlockSpec((H0, H1), lambda i: (0, 0)),
        pl.BlockSpec((1, H1), lambda i: (0, 0)),
        pl.BlockSpec((H1, H2), lambda i: (0, 0)),
        pl.BlockSpec((1, H2), lambda i: (0, 0)),
    ]
    args = [u_emb, i_emb, ubg, ibg, W0, b0, W1, b1, W2, b2]
    body = _mlp_body
    aliases = {}
    if out_buf is not None:
        in_specs.append(pl.BlockSpec(memory_space=pltpu.MemorySpace.HBM))
        args.append(out_buf)
        aliases = {10: 0}
        body = lambda *refs: _mlp_body(*refs[:10], refs[11])
    return pl.pallas_call(
        body,
        grid=(chb // BM,),
        in_specs=in_specs,
        out_specs=pl.BlockSpec((BM, H2), lambda i: (i + noff, 0)),
        out_shape=jax.ShapeDtypeStruct((B, H2), jnp.float32),
        input_output_aliases=aliases,
        compiler_params=pltpu.CompilerParams(
            dimension_semantics=("arbitrary",)),
    )(*args)


def kernel(user, item, user_table, item_table, user_bias, item_bias,
           W0, b0, W1, b1, W2, b2):
    offs = [sum(CHS[:c]) for c in range(len(CHS))]
    u0, i0, ubg, ibg = _sc_gather_chunk0_bias(
        user, item, user_table, item_table,
        user_bias.reshape(-1), item_bias.reshape(-1), CHS[0])
    embs = [(u0, i0)]
    embs += [_sc_gather_chunk(user, item, user_table, item_table, off, chb)
             for off, chb in zip(offs[1:], CHS[1:])]
    b0r, b1r, b2r = b0.reshape(1, H0), b1.reshape(1, H1), b2.reshape(1, H2)
    out = None
    for (off, chb), (u_emb, i_emb) in zip(zip(offs, CHS), embs):
        out = _mlp_chunk(off, chb, out, u_emb, i_emb, ubg, ibg,
                         W0, b0r, W1, b1r, W2, b2r)
    return out


# speedup vs baseline: 1.0960x; 1.0960x over previous
import functools

import jax
import jax.numpy as jnp
from jax import lax
from jax.experimental import pallas as pl
from jax.experimental.pallas import tpu as pltpu
from jax.experimental.pallas import tpu_sc as plsc

B = 16384
EMB = 128
H0, H1, H2 = 1024, 1024, 512
NC, NS = 2, 16
NW = NC * NS
CHS = (4096, 12288)
BM = 1024


def _sc_gather_chunk(user, item, user_table, item_table, off, chb):
    f32 = jnp.float32
    rpt = chb // NW
    mesh = plsc.VectorSubcoreMesh(core_axis_name="c", subcore_axis_name="s")
    out_type = (
        jax.ShapeDtypeStruct((chb, EMB), f32),
        jax.ShapeDtypeStruct((chb, EMB), f32),
    )

    @functools.partial(
        pl.kernel, mesh=mesh, out_type=out_type,
        scratch_types=[
            pltpu.VMEM((rpt,), jnp.int32),
            pltpu.VMEM((rpt,), jnp.int32),
            pltpu.VMEM((rpt, EMB), f32),
            pltpu.VMEM((rpt, EMB), f32),
            pltpu.SemaphoreType.DMA,
            pltpu.SemaphoreType.DMA,
            pltpu.SemaphoreType.DMA,
        ],
    )
    def k(u_hbm, i_hbm, ut_hbm, it_hbm, uo_hbm, io_hbm,
          uidx_v, iidx_v, urows_v, irows_v, gsem, osem0, osem1):
        wid = lax.axis_index("s") * NC + lax.axis_index("c")
        src = off + wid * rpt
        dst = wid * rpt
        pltpu.sync_copy(u_hbm.at[pl.ds(src, rpt)], uidx_v)
        pltpu.sync_copy(i_hbm.at[pl.ds(src, rpt)], iidx_v)
        pltpu.async_copy(ut_hbm.at[uidx_v], urows_v, gsem).wait()
        ou = pltpu.async_copy(urows_v, uo_hbm.at[pl.ds(dst, rpt)], osem0)
        pltpu.async_copy(it_hbm.at[iidx_v], irows_v, gsem).wait()
        oi = pltpu.async_copy(irows_v, io_hbm.at[pl.ds(dst, rpt)], osem1)
        ou.wait()
        oi.wait()

    return k(user, item, user_table, item_table)


def _sc_gather_chunk0_bias(user, item, user_table, item_table,
                           ub_flat, ib_flat, chb):
    f32 = jnp.float32
    rpt = chb // NW
    BPW = B // NW
    mesh = plsc.VectorSubcoreMesh(core_axis_name="c", subcore_axis_name="s")
    out_type = (
        jax.ShapeDtypeStruct((chb, EMB), f32),
        jax.ShapeDtypeStruct((chb, EMB), f32),
        jax.ShapeDtypeStruct((B,), f32),
        jax.ShapeDtypeStruct((B,), f32),
    )

    @functools.partial(
        pl.kernel, mesh=mesh, out_type=out_type,
        scratch_types=[
            pltpu.VMEM((rpt,), jnp.int32),
            pltpu.VMEM((rpt,), jnp.int32),
            pltpu.VMEM((rpt, EMB), f32),
            pltpu.VMEM((rpt, EMB), f32),
            pltpu.VMEM((BPW,), jnp.int32),
            pltpu.VMEM((BPW,), jnp.int32),
            pltpu.VMEM((BPW,), f32),
            pltpu.VMEM((BPW,), f32),
            pltpu.SemaphoreType.DMA,
            pltpu.SemaphoreType.DMA,
            pltpu.SemaphoreType.DMA,
            pltpu.SemaphoreType.DMA,
            pltpu.SemaphoreType.DMA,
        ],
    )
    def k(u_hbm, i_hbm, ut_hbm, it_hbm, ub_hbm, ib_hbm,
          uo_hbm, io_hbm, ubo_hbm, ibo_hbm,
          uidx_v, iidx_v, urows_v, irows_v,
          buidx_v, biidx_v, ubvals_v, ibvals_v,
          gsem, osem0, osem1, bsem0, bsem1):
        wid = lax.axis_index("s") * NC + lax.axis_index("c")
        src = wid * rpt
        base = wid * BPW
        pltpu.sync_copy(u_hbm.at[pl.ds(base, BPW)], buidx_v)
        pltpu.sync_copy(i_hbm.at[pl.ds(base, BPW)], biidx_v)
        bu = pltpu.async_copy(ub_hbm.at[buidx_v], ubvals_v, bsem0)
        bi = pltpu.async_copy(ib_hbm.at[biidx_v], ibvals_v, bsem1)
        pltpu.sync_copy(u_hbm.at[pl.ds(src, rpt)], uidx_v)
        pltpu.sync_copy(i_hbm.at[pl.ds(src, rpt)], iidx_v)
        pltpu.async_copy(ut_hbm.at[uidx_v], urows_v, gsem).wait()
        ou = pltpu.async_copy(urows_v, uo_hbm.at[pl.ds(src, rpt)], osem0)
        pltpu.async_copy(it_hbm.at[iidx_v], irows_v, gsem).wait()
        oi = pltpu.async_copy(irows_v, io_hbm.at[pl.ds(src, rpt)], osem1)
        bu.wait()
        ou2 = pltpu.async_copy(ubvals_v, ubo_hbm.at[pl.ds(base, BPW)], bsem0)
        bi.wait()
        bi2 = pltpu.async_copy(ibvals_v, ibo_hbm.at[pl.ds(base, BPW)], bsem1)
        ou.wait()
        oi.wait()
        ou2.wait()
        bi2.wait()

    return k(user, item, user_table, item_table, ub_flat, ib_flat)


def _mlp_body(u_ref, i_ref, ub_ref, ib_ref, w0_ref, b0_ref, w1_ref, b1_ref,
              w2_ref, b2_ref, o_ref):
    f32 = jnp.float32
    w0 = w0_ref[...]
    w0u, w0i = w0[:EMB], w0[EMB:]
    x = jnp.concatenate([u_ref[...], i_ref[...]], axis=1)
    acc = jnp.dot(x, w0, preferred_element_type=f32)
    su = jnp.sum(w0u, axis=0, keepdims=True)
    si = jnp.sum(w0i, axis=0, keepdims=True)
    ub = ub_ref[...].reshape(BM, 1)
    ib = ib_ref[...].reshape(BM, 1)
    acc = acc + b0_ref[...] + ub * su + ib * si
    h = jnp.maximum(acc, 0.0)
    h = jnp.maximum(
        jnp.dot(h, w1_ref[...], preferred_element_type=f32) + b1_ref[...],
        0.0)
    o_ref[...] = jnp.maximum(
        jnp.dot(h, w2_ref[...], preferred_element_type=f32) + b2_ref[...],
        0.0)


def _mlp_chunk(off, chb, out_buf, u_emb, i_emb, ubg, ibg,
               W0, b0, W1, b1, W2, b2):
    noff = off // BM
    in_specs = [
        pl.BlockSpec((BM, EMB), lambda i: (i, 0)),
        pl.BlockSpec((BM, EMB), lambda i: (i, 0)),
        pl.BlockSpec((BM,), lambda i: (i + noff,)),
        pl.BlockSpec((BM,), lambda i: (i + noff,)),
        pl.BlockSpec((2 * EMB, H0), lambda i: (0, 0)),
        pl.BlockSpec((1, H0), lambda i: (0, 0)),
        pl.BlockSpec((H0, H1), lambda i: (0, 0)),
        pl.BlockSpec((1, H1), lambda i: (0, 0)),
        pl.BlockSpec((H1, H2), lambda i: (0, 0)),
        pl.BlockSpec((1, H2), lambda i: (0, 0)),
    ]
    args = [u_emb, i_emb, ubg, ibg, W0, b0, W1, b1, W2, b2]
    body = _mlp_body
    aliases = {}
    if out_buf is not None:
        in_specs.append(pl.BlockSpec(memory_space=pltpu.MemorySpace.HBM))
        args.append(out_buf)
        aliases = {10: 0}
        body = lambda *refs: _mlp_body(*refs[:10], refs[11])
    return pl.pallas_call(
        body,
        grid=(chb // BM,),
        in_specs=in_specs,
        out_specs=pl.BlockSpec((BM, H2), lambda i: (i + noff, 0)),
        out_shape=jax.ShapeDtypeStruct((B, H2), jnp.float32),
        input_output_aliases=aliases,
        compiler_params=pltpu.CompilerParams(
            dimension_semantics=("arbitrary",)),
    )(*args)


def kernel(user, item, user_table, item_table, user_bias, item_bias,
           W0, b0, W1, b1, W2, b2):
    offs = [sum(CHS[:c]) for c in range(len(CHS))]
    u0, i0, ubg, ibg = _sc_gather_chunk0_bias(
        user, item, user_table, item_table,
        user_bias.reshape(-1), item_bias.reshape(-1), CHS[0])
    embs = [(u0, i0)]
    embs += [_sc_gather_chunk(user, item, user_table, item_table, off, chb)
             for off, chb in zip(offs[1:], CHS[1:])]
    b0r, b1r, b2r = b0.reshape(1, H0), b1.reshape(1, H1), b2.reshape(1, H2)
    out = None
    for (off, chb), (u_emb, i_emb) in zip(zip(offs, CHS), embs):
        out = _mlp_chunk(off, chb, out, u_emb, i_emb, ubg, ibg,
                         W0, b0r, W1, b1r, W2, b2r)
    return out
